# baseline (speedup 1.0000x reference)
import functools

import jax
import jax.numpy as jnp
from jax import lax
from jax.experimental import pallas as pl
from jax.experimental.pallas import tpu as pltpu
from jax.experimental.pallas import tpu_sc as plsc

_VOCAB = 1000000
_CLASSES = 1000
_D = 32
_B = 4096
_L = 200

_NC = 2
_NS = 16
_NW = _NC * _NS
_ROWS_PER_W = _B // _NW
_HALF = _L // 2
_NBUF = 4


_NTILE_FULL = 7812
_TPW = _NTILE_FULL // _NW
_NB1 = 8
_TMAIN = (_TPW // _NB1) * _NB1


def _sc_fmt_body(m1_hbm, tail_hbm, out_hbm, in_bufs, ob0, ob1, ob2, ob3, ob4, ob5, ob6, ob7, in_sems, out_sems):
    wid = lax.axis_index("s") * _NC + lax.axis_index("c")
    base_t = wid * _TPW
    outs = (ob0, ob1, ob2, ob3, ob4, ob5, ob6, ob7)

    def fire_in(t, s):
        pltpu.async_copy(
            m1_hbm.at[pl.ds(t * 32, 32), :],
            in_bufs.at[s],
            in_sems.at[s],
        )

    def wait_in(t, s):
        pltpu.make_async_copy(
            m1_hbm.at[pl.ds(t * 32, 32), :],
            in_bufs.at[s],
            in_sems.at[s],
        ).wait()

    def fire_out(t, s):
        pltpu.async_copy(
            outs[s], out_hbm.at[pl.ds(t * 4096, 4096)], out_sems.at[s]
        )

    def wait_out(t, s):
        pltpu.make_async_copy(
            outs[s], out_hbm.at[pl.ds(t * 4096, 4096)], out_sems.at[s]
        ).wait()

    iota32 = lax.iota(jnp.int32, 16) * 32

    def transpose_tile(s, nj):
        def jgbody(jg, carry):
            base = iota32 + jg * 512
            for d in range(_D):
                v = in_bufs[s, d, pl.ds(jg * 16, 16)]
                plsc.store_scatter(outs[s], [base + d], v)
            return carry

        lax.fori_loop(0, nj // 16, jgbody, 0, unroll=2)

    for s in range(_NB1):
        fire_in(base_t + s, s)

    def outer(ti, carry):
        for s in range(_NB1):
            t = base_t + ti * _NB1 + s
            wait_in(t, s)

            @pl.when(ti > 0)
            def _():
                wait_out(t - _NB1, s)

            transpose_tile(s, 128)
            fire_out(t, s)

            @pl.when(ti < (_TPW // _NB1) - 1)
            def _():
                fire_in(t + _NB1, s)

        return carry

    lax.fori_loop(0, _TMAIN // _NB1, outer, 0)
    for s in range(_NB1):
        wait_out(base_t + _TMAIN - _NB1 + s, s)

    for k in range(_TPW - _TMAIN):
        t = base_t + _TMAIN + k
        fire_in(t, k)
        wait_in(t, k)
        transpose_tile(k, 128)
        fire_out(t, k)
        wait_out(t, k)

    @pl.when(wid < 4)
    def _():
        t = _NTILE_FULL - 4 + wid
        fire_in(t, 0)
        wait_in(t, 0)
        transpose_tile(0, 128)
        fire_out(t, 0)
        wait_out(t, 0)

    @pl.when(wid == 31)
    def _():
        pltpu.sync_copy(tail_hbm, ob0.at[pl.ds(0, 64 * _D)])
        pltpu.sync_copy(
            ob0.at[pl.ds(0, 64 * _D)],
            out_hbm.at[pl.ds((_NTILE_FULL * 128) * _D, 64 * _D)],
        )


@functools.cache
def _sc_fmt():
    return pl.kernel(
        _sc_fmt_body,
        mesh=plsc.VectorSubcoreMesh(core_axis_name="c", subcore_axis_name="s"),
        compiler_params=pltpu.CompilerParams(
            use_tc_tiling_on_sc=False, needs_layout_passes=False
        ),
        out_type=jax.ShapeDtypeStruct((_VOCAB * _D,), jnp.float32),
        scratch_types=[
            pltpu.VMEM((_NB1, _D, 128), jnp.float32),
            pltpu.VMEM((128 * _D,), jnp.float32),
            pltpu.VMEM((128 * _D,), jnp.float32),
            pltpu.VMEM((128 * _D,), jnp.float32),
            pltpu.VMEM((128 * _D,), jnp.float32),
            pltpu.VMEM((128 * _D,), jnp.float32),
            pltpu.VMEM((128 * _D,), jnp.float32),
            pltpu.VMEM((128 * _D,), jnp.float32),
            pltpu.VMEM((128 * _D,), jnp.float32),

            pltpu.SemaphoreType.DMA((_NB1,)),
            pltpu.SemaphoreType.DMA((_NB1,)),
        ],
    )


def _sc_pool_body(x_hbm, table_hbm, out_hbm, idx_v, bufs, acc, sems):
    wid = lax.axis_index("s") * _NC + lax.axis_index("c")
    row_base = wid * _ROWS_PER_W

    pltpu.sync_copy(x_hbm.at[pl.ds(row_base, _ROWS_PER_W)], idx_v)

    def fire(b, s):
        pltpu.async_copy(
            table_hbm.at[idx_v.at[b]],
            bufs.at[s],
            sems.at[s],
        )

    def drain(b, s):
        pltpu.make_async_copy(
            table_hbm.at[idx_v.at[b]],
            bufs.at[s],
            sems.at[s],
        ).wait()

    for s in range(_NBUF):
        fire(s, s)

    zeros = jnp.zeros((16,), jnp.float32)

    def outer(bb, carry):
        for s in range(_NBUF):
            b = bb * _NBUF + s
            drain(b, s)

            def body(r, c):
                a0, a1, a2, a3 = c
                a0 = a0 + bufs[s, 2 * r, pl.ds(0, 16)]
                a1 = a1 + bufs[s, 2 * r, pl.ds(16, 16)]
                a2 = a2 + bufs[s, 2 * r + 1, pl.ds(0, 16)]
                a3 = a3 + bufs[s, 2 * r + 1, pl.ds(16, 16)]
                return (a0, a1, a2, a3)

            nb = b + _NBUF

            @pl.when(nb < _ROWS_PER_W)
            def _():
                fire(nb, s)

            a0, a1, a2, a3 = lax.fori_loop(
                0, _L // 2, body, (zeros, zeros, zeros, zeros), unroll=2
            )
            acc[b, pl.ds(0, 16)] = a0 + a2
            acc[b, pl.ds(16, 16)] = a1 + a3
        return carry

    lax.fori_loop(0, _ROWS_PER_W // _NBUF, outer, 0)

    pltpu.sync_copy(acc, out_hbm.at[pl.ds(row_base, _ROWS_PER_W)])


@functools.cache
def _sc_pool():
    return pl.kernel(
        _sc_pool_body,
        mesh=plsc.VectorSubcoreMesh(core_axis_name="c", subcore_axis_name="s"),
        compiler_params=pltpu.CompilerParams(use_tc_tiling_on_sc=False),
        out_type=jax.ShapeDtypeStruct((_B, _D), jnp.float32),
        scratch_types=[
            pltpu.VMEM((_ROWS_PER_W, _L), jnp.int32),
            pltpu.VMEM((_NBUF, _L, _D), jnp.float32),
            pltpu.VMEM((_ROWS_PER_W, _D), jnp.float32),
            pltpu.SemaphoreType.DMA((_NBUF,)),
        ],
    )


_F1G = 489


def _tc_fmt1_body(tt_ref, o_ref):
    x = tt_ref[...]
    o_ref[...] = jnp.concatenate(
        [x[:, 128 * k : 128 * (k + 1)] for k in range(16)], axis=0
    )


def _tc_fmt1(tt):
    return pl.pallas_call(
        _tc_fmt1_body,
        grid=(_F1G,),
        in_specs=[pl.BlockSpec((_D, 2048), lambda i: (0, i))],
        out_specs=pl.BlockSpec((512, 128), lambda i: (i, 0)),
        out_shape=jax.ShapeDtypeStruct((_F1G * 512, 128), jnp.float32),
    )(tt)


def _tc_head_body(p_ref, w_ref, b_ref, o_ref):
    t = jnp.tanh(p_ref[...] * (1.0 / _L))
    o_ref[...] = (
        lax.dot_general(
            t, w_ref[...], (((1,), (1,)), ((), ())),
            preferred_element_type=jnp.float32,
        )
        + b_ref[...]
    )


def _tc_head(pooled, W, b2d):
    blk = 512
    return pl.pallas_call(
        _tc_head_body,
        grid=(_B // blk,),
        in_specs=[
            pl.BlockSpec((blk, _D), lambda i: (i, 0)),
            pl.BlockSpec((_CLASSES, _D), lambda i: (0, 0)),
            pl.BlockSpec((1, _CLASSES), lambda i: (0, 0)),
        ],
        out_specs=pl.BlockSpec((blk, _CLASSES), lambda i: (i, 0)),
        out_shape=jax.ShapeDtypeStruct((_B, _CLASSES), jnp.float32),
    )(pooled, W, b2d)


@jax.jit
def kernel(x, emb_table, W, b):
    tail = emb_table[_NTILE_FULL * 128 :, :].reshape(64 * _D)
    m1 = _tc_fmt1(emb_table.T)
    t_lin = _sc_fmt()(m1, tail)
    pooled = _sc_pool()(x, t_lin.reshape(_VOCAB, _D))
    return _tc_head(pooled, W, b.reshape(1, _CLASSES))


# speedup vs baseline: 1.1416x; 1.1416x over previous
import functools

import jax
import jax.numpy as jnp
from jax import lax
from jax.experimental import pallas as pl
from jax.experimental.pallas import tpu as pltpu
from jax.experimental.pallas import tpu_sc as plsc

_VOCAB = 1000000
_CLASSES = 1000
_D = 32
_B = 4096
_L = 200

_NC = 2
_NS = 16
_NW = _NC * _NS
_ROWS_PER_W = _B // _NW
_HALF = _L // 2
_NBUF = 4


_NTILE_FULL = 7812
_TPW = _NTILE_FULL // _NW
_NB1 = 8
_TMAIN = (_TPW // _NB1) * _NB1


def _sc_fmt_body(m1_hbm, tail_hbm, out_hbm, in_bufs, ob0, ob1, ob2, ob3, ob4, ob5, ob6, ob7, in_sems, out_sems):
    wid = lax.axis_index("s") * _NC + lax.axis_index("c")
    base_t = wid * _TPW
    outs = (ob0, ob1, ob2, ob3, ob4, ob5, ob6, ob7)

    def fire_in(t, s):
        pltpu.async_copy(
            m1_hbm.at[pl.ds(t * 32, 32), :],
            in_bufs.at[s],
            in_sems.at[s],
        )

    def wait_in(t, s):
        pltpu.make_async_copy(
            m1_hbm.at[pl.ds(t * 32, 32), :],
            in_bufs.at[s],
            in_sems.at[s],
        ).wait()

    def fire_out(t, s):
        pltpu.async_copy(
            outs[s], out_hbm.at[pl.ds(t * 4096, 4096)], out_sems.at[s]
        )

    def wait_out(t, s):
        pltpu.make_async_copy(
            outs[s], out_hbm.at[pl.ds(t * 4096, 4096)], out_sems.at[s]
        ).wait()

    iota32 = lax.iota(jnp.int32, 16) * 32

    def transpose_tile(s, nj):
        def jgbody(jg, carry):
            base = iota32 + jg * 512
            for d in range(_D):
                v = in_bufs[s, d, pl.ds(jg * 16, 16)]
                plsc.store_scatter(outs[s], [base + d], v)
            return carry

        lax.fori_loop(0, nj // 16, jgbody, 0, unroll=2)

    for s in range(_NB1):
        fire_in(base_t + s, s)

    def outer(ti, carry):
        for s in range(_NB1):
            t = base_t + ti * _NB1 + s
            wait_in(t, s)

            @pl.when(ti > 0)
            def _():
                wait_out(t - _NB1, s)

            transpose_tile(s, 128)
            fire_out(t, s)

            @pl.when(ti < (_TPW // _NB1) - 1)
            def _():
                fire_in(t + _NB1, s)

        return carry

    lax.fori_loop(0, _TMAIN // _NB1, outer, 0)
    for s in range(_NB1):
        wait_out(base_t + _TMAIN - _NB1 + s, s)

    for k in range(_TPW - _TMAIN):
        t = base_t + _TMAIN + k
        fire_in(t, k)
        wait_in(t, k)
        transpose_tile(k, 128)
        fire_out(t, k)
        wait_out(t, k)

    @pl.when(wid < 4)
    def _():
        t = _NTILE_FULL - 4 + wid
        fire_in(t, 0)
        wait_in(t, 0)
        transpose_tile(0, 128)
        fire_out(t, 0)
        wait_out(t, 0)

    @pl.when(wid == 31)
    def _():
        pltpu.sync_copy(tail_hbm, ob0.at[pl.ds(0, 64 * _D)])
        pltpu.sync_copy(
            ob0.at[pl.ds(0, 64 * _D)],
            out_hbm.at[pl.ds((_NTILE_FULL * 128) * _D, 64 * _D)],
        )


@functools.cache
def _sc_fmt():
    return pl.kernel(
        _sc_fmt_body,
        mesh=plsc.VectorSubcoreMesh(core_axis_name="c", subcore_axis_name="s"),
        compiler_params=pltpu.CompilerParams(
            use_tc_tiling_on_sc=False, needs_layout_passes=False
        ),
        out_type=jax.ShapeDtypeStruct((_VOCAB * _D,), jnp.float32),
        scratch_types=[
            pltpu.VMEM((_NB1, _D, 128), jnp.float32),
            pltpu.VMEM((128 * _D,), jnp.float32),
            pltpu.VMEM((128 * _D,), jnp.float32),
            pltpu.VMEM((128 * _D,), jnp.float32),
            pltpu.VMEM((128 * _D,), jnp.float32),
            pltpu.VMEM((128 * _D,), jnp.float32),
            pltpu.VMEM((128 * _D,), jnp.float32),
            pltpu.VMEM((128 * _D,), jnp.float32),
            pltpu.VMEM((128 * _D,), jnp.float32),

            pltpu.SemaphoreType.DMA((_NB1,)),
            pltpu.SemaphoreType.DMA((_NB1,)),
        ],
    )


def _sc_pool_body(x_hbm, table_hbm, out_hbm, idx_v, bufs, acc, sems):
    wid = lax.axis_index("s") * _NC + lax.axis_index("c")
    row_base = wid * _ROWS_PER_W

    pltpu.sync_copy(x_hbm.at[pl.ds(row_base, _ROWS_PER_W)], idx_v)

    def fire(b, s):
        pltpu.async_copy(
            table_hbm.at[idx_v.at[b]],
            bufs.at[s],
            sems.at[s],
        )

    def drain(b, s):
        pltpu.make_async_copy(
            table_hbm.at[idx_v.at[b]],
            bufs.at[s],
            sems.at[s],
        ).wait()

    for s in range(_NBUF):
        fire(s, s)

    zeros = jnp.zeros((16,), jnp.float32)

    def outer(bb, carry):
        for s in range(_NBUF):
            b = bb * _NBUF + s
            drain(b, s)

            def body(r, c):
                a0, a1, a2, a3 = c
                a0 = a0 + bufs[s, 2 * r, pl.ds(0, 16)]
                a1 = a1 + bufs[s, 2 * r, pl.ds(16, 16)]
                a2 = a2 + bufs[s, 2 * r + 1, pl.ds(0, 16)]
                a3 = a3 + bufs[s, 2 * r + 1, pl.ds(16, 16)]
                return (a0, a1, a2, a3)

            nb = b + _NBUF

            @pl.when(nb < _ROWS_PER_W)
            def _():
                fire(nb, s)

            a0, a1, a2, a3 = lax.fori_loop(
                0, _L // 2, body, (zeros, zeros, zeros, zeros), unroll=2
            )
            acc[b, pl.ds(0, 16)] = a0 + a2
            acc[b, pl.ds(16, 16)] = a1 + a3
        return carry

    lax.fori_loop(0, _ROWS_PER_W // _NBUF, outer, 0)

    pltpu.sync_copy(acc, out_hbm.at[pl.ds(row_base, _ROWS_PER_W)])


@functools.cache
def _sc_pool():
    return pl.kernel(
        _sc_pool_body,
        mesh=plsc.VectorSubcoreMesh(core_axis_name="c", subcore_axis_name="s"),
        compiler_params=pltpu.CompilerParams(use_tc_tiling_on_sc=False),
        out_type=jax.ShapeDtypeStruct((_B, _D), jnp.float32),
        scratch_types=[
            pltpu.VMEM((_ROWS_PER_W, _L), jnp.int32),
            pltpu.VMEM((_NBUF, _L, _D), jnp.float32),
            pltpu.VMEM((_ROWS_PER_W, _D), jnp.float32),
            pltpu.SemaphoreType.DMA((_NBUF,)),
        ],
    )


_F1G = 489


def _tc_fmt_body(tt_ref, eye_ref, o_ref):
    o_ref[...] = lax.dot_general(
        tt_ref[...], eye_ref[...], (((0,), (0,)), ((), ())),
        preferred_element_type=jnp.float32,
    )


def _tc_fmt(tt, eye):
    return pl.pallas_call(
        _tc_fmt_body,
        grid=(_F1G,),
        in_specs=[
            pl.BlockSpec((_D, 2048), lambda i: (0, i)),
            pl.BlockSpec((_D, _D), lambda i: (0, 0)),
        ],
        out_specs=pl.BlockSpec((2048, _D), lambda i: (i, 0)),
        out_shape=jax.ShapeDtypeStruct((_F1G * 2048, _D), jnp.float32),
    )(tt, eye)


def _tc_head_body(p_ref, w_ref, b_ref, o_ref):
    t = jnp.tanh(p_ref[...] * (1.0 / _L))
    o_ref[...] = (
        lax.dot_general(
            t, w_ref[...], (((1,), (1,)), ((), ())),
            preferred_element_type=jnp.float32,
        )
        + b_ref[...]
    )


def _tc_head(pooled, W, b2d):
    blk = 512
    return pl.pallas_call(
        _tc_head_body,
        grid=(_B // blk,),
        in_specs=[
            pl.BlockSpec((blk, _D), lambda i: (i, 0)),
            pl.BlockSpec((_CLASSES, _D), lambda i: (0, 0)),
            pl.BlockSpec((1, _CLASSES), lambda i: (0, 0)),
        ],
        out_specs=pl.BlockSpec((blk, _CLASSES), lambda i: (i, 0)),
        out_shape=jax.ShapeDtypeStruct((_B, _CLASSES), jnp.float32),
    )(pooled, W, b2d)


@jax.jit
def kernel(x, emb_table, W, b):
    t2 = _tc_fmt(emb_table.T, jnp.eye(_D, dtype=jnp.float32))
    pooled = _sc_pool()(x, t2)
    return _tc_head(pooled, W, b.reshape(1, _CLASSES))


# speedup vs baseline: 1.7555x; 1.5378x over previous
import functools

import jax
import jax.numpy as jnp
from jax import lax
from jax.experimental import pallas as pl
from jax.experimental.pallas import tpu as pltpu
from jax.experimental.pallas import tpu_sc as plsc

_VOCAB = 1000000
_CLASSES = 1000
_D = 32
_B = 4096
_L = 200

_NC = 2
_NS = 16
_NW = _NC * _NS
_ROWS_PER_W = _B // _NW
_NBUF = 4


def _sc_pool_body(x_hbm, table_hbm, out_hbm, idx_v, bufs, acc, sems):
    wid = lax.axis_index("s") * _NC + lax.axis_index("c")
    row_base = wid * _ROWS_PER_W

    pltpu.sync_copy(x_hbm.at[pl.ds(row_base, _ROWS_PER_W)], idx_v)

    def fire(b, s):
        pltpu.async_copy(
            table_hbm.at[idx_v.at[b]],
            bufs.at[s],
            sems.at[s],
        )

    def drain(b, s):
        pltpu.make_async_copy(
            table_hbm.at[idx_v.at[b]],
            bufs.at[s],
            sems.at[s],
        ).wait()

    for s in range(_NBUF):
        fire(s, s)

    zeros = jnp.zeros((16,), jnp.float32)

    def outer(bb, carry):
        for s in range(_NBUF):
            b = bb * _NBUF + s
            drain(b, s)

            def body(r, c):
                a0, a1, a2, a3 = c
                a0 = a0 + bufs[s, 2 * r, pl.ds(0, 16)]
                a1 = a1 + bufs[s, 2 * r, pl.ds(16, 16)]
                a2 = a2 + bufs[s, 2 * r + 1, pl.ds(0, 16)]
                a3 = a3 + bufs[s, 2 * r + 1, pl.ds(16, 16)]
                return (a0, a1, a2, a3)

            nb = b + _NBUF

            @pl.when(nb < _ROWS_PER_W)
            def _():
                fire(nb, s)

            a0, a1, a2, a3 = lax.fori_loop(
                0, _L // 2, body, (zeros, zeros, zeros, zeros), unroll=2
            )
            acc[b, pl.ds(0, 16)] = a0 + a2
            acc[b, pl.ds(16, 16)] = a1 + a3
        return carry

    lax.fori_loop(0, _ROWS_PER_W // _NBUF, outer, 0)

    pltpu.sync_copy(acc, out_hbm.at[pl.ds(row_base, _ROWS_PER_W)])


@functools.cache
def _sc_pool():
    return pl.kernel(
        _sc_pool_body,
        mesh=plsc.VectorSubcoreMesh(core_axis_name="c", subcore_axis_name="s"),
        compiler_params=pltpu.CompilerParams(use_tc_tiling_on_sc=False),
        out_type=jax.ShapeDtypeStruct((_B, _D), jnp.float32),
        scratch_types=[
            pltpu.VMEM((_ROWS_PER_W, _L), jnp.int32),
            pltpu.VMEM((_NBUF, _L, _D), jnp.float32),
            pltpu.VMEM((_ROWS_PER_W, _D), jnp.float32),
            pltpu.SemaphoreType.DMA((_NBUF,)),
        ],
    )


def _tc_head_body(p_ref, w_ref, b_ref, o_ref):
    t = jnp.tanh(p_ref[...] * (1.0 / _L))
    o_ref[...] = (
        lax.dot_general(
            t, w_ref[...], (((1,), (1,)), ((), ())),
            preferred_element_type=jnp.float32,
        )
        + b_ref[...]
    )


def _tc_head(pooled, W, b2d):
    blk = 512
    return pl.pallas_call(
        _tc_head_body,
        grid=(_B // blk,),
        in_specs=[
            pl.BlockSpec((blk, _D), lambda i: (i, 0)),
            pl.BlockSpec((_CLASSES, _D), lambda i: (0, 0)),
            pl.BlockSpec((1, _CLASSES), lambda i: (0, 0)),
        ],
        out_specs=pl.BlockSpec((blk, _CLASSES), lambda i: (i, 0)),
        out_shape=jax.ShapeDtypeStruct((_B, _CLASSES), jnp.float32),
    )(pooled, W, b2d)


@jax.jit
def kernel(x, emb_table, W, b):
    pooled = _sc_pool()(x, emb_table)
    return _tc_head(pooled, W, b.reshape(1, _CLASSES))


# speedup vs baseline: 1.8023x; 1.0267x over previous
import functools

import jax
import jax.numpy as jnp
from jax import lax
from jax.experimental import pallas as pl
from jax.experimental.pallas import tpu as pltpu
from jax.experimental.pallas import tpu_sc as plsc

_VOCAB = 1000000
_CLASSES = 1000
_D = 32
_B = 4096
_L = 200

_NC = 2
_NS = 16
_NW = _NC * _NS
_ROWS_PER_W = _B // _NW
_NBUF = 4


def _sc_pool_body(x_hbm, table_hbm, out_hbm, idx_v, bufs, acc, sems):
    wid = lax.axis_index("s") * _NC + lax.axis_index("c")
    row_base = wid * _ROWS_PER_W

    pltpu.sync_copy(x_hbm.at[pl.ds(row_base, _ROWS_PER_W)], idx_v)

    def fire(b, s):
        pltpu.async_copy(
            table_hbm.at[idx_v.at[b]],
            bufs.at[s],
            sems.at[s],
        )

    def drain(b, s):
        pltpu.make_async_copy(
            table_hbm.at[idx_v.at[b]],
            bufs.at[s],
            sems.at[s],
        ).wait()

    for s in range(_NBUF):
        fire(s, s)

    zeros = jnp.zeros((16,), jnp.float32)

    def outer(bb, carry):
        for s in range(_NBUF):
            b = bb * _NBUF + s
            drain(b, s)

            def body(r, c):
                a0, a1, a2, a3 = c
                a0 = a0 + bufs[s, 2 * r, pl.ds(0, 16)]
                a1 = a1 + bufs[s, 2 * r, pl.ds(16, 16)]
                a2 = a2 + bufs[s, 2 * r + 1, pl.ds(0, 16)]
                a3 = a3 + bufs[s, 2 * r + 1, pl.ds(16, 16)]
                return (a0, a1, a2, a3)

            nb = b + _NBUF

            @pl.when(nb < _ROWS_PER_W)
            def _():
                fire(nb, s)

            a0, a1, a2, a3 = lax.fori_loop(
                0, _L // 2, body, (zeros, zeros, zeros, zeros), unroll=2
            )
            acc[b, pl.ds(0, 16)] = a0 + a2
            acc[b, pl.ds(16, 16)] = a1 + a3
        return carry

    lax.fori_loop(0, _ROWS_PER_W // _NBUF, outer, 0)

    pltpu.sync_copy(acc, out_hbm.at[pl.ds(row_base, _ROWS_PER_W)])


@functools.cache
def _sc_pool():
    return pl.kernel(
        _sc_pool_body,
        mesh=plsc.VectorSubcoreMesh(core_axis_name="c", subcore_axis_name="s"),
        compiler_params=pltpu.CompilerParams(use_tc_tiling_on_sc=False),
        out_type=jax.ShapeDtypeStruct((_B, _D), jnp.float32),
        scratch_types=[
            pltpu.VMEM((_ROWS_PER_W, _L), jnp.int32),
            pltpu.VMEM((_NBUF, _L, _D), jnp.float32),
            pltpu.VMEM((_ROWS_PER_W, _D), jnp.float32),
            pltpu.SemaphoreType.DMA((_NBUF,)),
        ],
    )


def _tc_head_body(p_ref, w_ref, b_ref, o_ref):
    t = jnp.tanh(p_ref[...] * (1.0 / _L))
    o_ref[...] = (
        lax.dot_general(
            w_ref[...], t, (((1,), (1,)), ((), ())),
            preferred_element_type=jnp.float32,
        )
        + b_ref[...]
    )


def _tc_head(pooled, W, b2d):
    blk = 512
    out_t = pl.pallas_call(
        _tc_head_body,
        grid=(_B // blk,),
        in_specs=[
            pl.BlockSpec((blk, _D), lambda i: (i, 0)),
            pl.BlockSpec((_CLASSES, _D), lambda i: (0, 0)),
            pl.BlockSpec((_CLASSES, 1), lambda i: (0, 0)),
        ],
        out_specs=pl.BlockSpec((_CLASSES, blk), lambda i: (0, i)),
        out_shape=jax.ShapeDtypeStruct((_CLASSES, _B), jnp.float32),
    )(pooled, W, b2d)
    return out_t.T


@jax.jit
def kernel(x, emb_table, W, b):
    pooled = _sc_pool()(x, emb_table)
    return _tc_head(pooled, W, b.reshape(_CLASSES, 1))
